# baseline (speedup 1.0000x reference)
import functools

import jax
import jax.numpy as jnp
from jax import lax
from jax.experimental import pallas as pl
from jax.experimental.pallas import tpu as pltpu
from jax.experimental.pallas import tpu_sc as plsc

VOCAB = 1000000
DIMS = 64
BATCH = 4096
N_CTX = 20

NC = 2
NS = 16
NW = NC * NS
CPW = BATCH // NW
GSZ = 128
NG = (CPW * N_CTX) // GSZ
LANES = 16
EPS = 1e-8


def _rsqrt_nr(x):
    i = lax.bitcast_convert_type(x, jnp.int32)
    i = 0x5F3759DF - (i >> 1)
    y = lax.bitcast_convert_type(i, jnp.float32)
    for _ in range(3):
        y = y * (1.5 - 0.5 * x * y * y)
    return y


def _make_kernel():
    mesh = plsc.VectorSubcoreMesh(core_axis_name="c", subcore_axis_name="s")

    @functools.partial(
        pl.kernel,
        mesh=mesh,
        compiler_params=pltpu.CompilerParams(
            needs_layout_passes=False, use_tc_tiling_on_sc=False),
        out_type=jax.ShapeDtypeStruct((NW, LANES), jnp.float32),
        scratch_types=[
            pltpu.VMEM((CPW,), jnp.int32),
            pltpu.VMEM((2 * NG, GSZ), jnp.int32),
            pltpu.VMEM((CPW, DIMS), jnp.float32),
            pltpu.VMEM((CPW,), jnp.float32),
            pltpu.VMEM((GSZ, DIMS), jnp.float32),
            pltpu.VMEM((LANES,), jnp.float32),
            pltpu.SemaphoreType.DMA,
        ],
    )
    def word2vec_loss(ctab, xtab, cidx_h, aidx_h, out_h,
                      cidx_v, aidx_v, ctr_v, cc_v, buf_v, out_v, sem):
        wid = lax.axis_index("s") * NC + lax.axis_index("c")
        lanes = lax.iota(jnp.int32, LANES)
        zeros = jnp.zeros((LANES,), jnp.float32)

        pltpu.sync_copy(cidx_h.at[wid], cidx_v)
        pltpu.sync_copy(aidx_h.at[wid], aidx_v)
        pltpu.async_copy(ctab.at[cidx_v], ctr_v, sem).wait()

        def cc_sub(sub, carry):
            cid = lanes + sub * LANES

            def kblk(kb, acc):
                for j in range(8):
                    col = lanes * 0 + (kb * 8 + j)
                    ck = plsc.load_gather(ctr_v, [cid, col])
                    acc = acc + ck * ck
                return acc

            acc = lax.fori_loop(0, 8, kblk, zeros)
            cc_v[pl.ds(sub * LANES, LANES)] = acc
            return carry

        lax.fori_loop(0, CPW // LANES, cc_sub, 0)

        def group(g, acc_g):
            pltpu.async_copy(xtab.at[aidx_v.at[g]], buf_v, sem).wait()
            base = lax.rem(g, NG) * GSZ
            gv = lanes * 0 + g

            def sub_body(sub, acc):
                rid = lanes + sub * LANES
                cid = (base + rid) // N_CTX
                ccr = plsc.load_gather(cc_v, [cid])

                def kblk(kb, carry):
                    cx, xx = carry
                    for j in range(8):
                        col = lanes * 0 + (kb * 8 + j)
                        xk = plsc.load_gather(buf_v, [rid, col])
                        ck = plsc.load_gather(ctr_v, [cid, col])
                        cx = cx + ck * xk
                        xx = xx + xk * xk
                    return (cx, xx)

                cx, xx = lax.fori_loop(0, 8, kblk, (zeros, zeros))
                d2 = xx * ccr
                nrm = d2 * _rsqrt_nr(jnp.maximum(d2, 1e-30))
                cos = cx / jnp.maximum(nrm, EPS)
                contrib = jnp.where(gv < NG, 1.0 - cos, jnp.maximum(cos, 0.0))
                return acc + contrib

            return lax.fori_loop(0, GSZ // LANES, sub_body, acc_g)

        acc = lax.fori_loop(0, 2 * NG, group, zeros)
        out_v[...] = acc * (1.0 / (BATCH * N_CTX))
        pltpu.sync_copy(out_v, out_h.at[wid])

    return word2vec_loss


_KERNEL = _make_kernel()


@jax.jit
def kernel(centers_table, contexts_table, center_idxs, context_idxs, neg_idxs):
    cidx = center_idxs.astype(jnp.int32).reshape(NW, CPW)
    xid = context_idxs.astype(jnp.int32).reshape(NW, NG, GSZ)
    nid = neg_idxs.astype(jnp.int32).reshape(NW, NG, GSZ)
    aidx = jnp.concatenate([xid, nid], axis=1)
    partials = _KERNEL(centers_table, contexts_table, cidx, aidx)
    return jnp.sum(partials)
